# baseline (speedup 1.0000x reference)
import functools

import jax
import jax.numpy as jnp
from jax import lax
from jax.experimental import pallas as pl
from jax.experimental.pallas import tpu as pltpu
from jax.experimental.pallas import tpu_sc as plsc

_MASK_PERCENT = 0.5
_NBUF = 2
_NLANES = 16


def _sort_asc(v):
    return lax.sort(v, dimension=0)


def _sort_desc(v):
    return lax.rev(lax.sort(v, dimension=0), (0,))


def _sc_body(r_hbm, x_hbm, o_hbm, r_v, m_v, xb_v, ob_v, in_sem, out_sem):
    freq = x_hbm.shape[0]
    length = x_hbm.shape[2]
    n2 = x_hbm.shape[3]
    cid = lax.axis_index("c")
    sid = lax.axis_index("s")
    wid = sid * 2 + cid
    g_per_w = x_hbm.shape[1] // 32
    gbase = wid * g_per_w
    cols = g_per_w * n2
    gc = g_per_w // 2
    slots = 2 * freq
    nfc = freq // _NLANES

    pltpu.sync_copy(r_hbm.at[:, pl.ds(gbase * n2, cols)], r_v)

    def mask_chunk(ch, carry):
        for cc in range(_NLANES):
            c = ch * _NLANES + cc
            cvec = jnp.full((_NLANES,), c, jnp.int32)
            ks = []
            for fc in range(nfc):
                fvec = fc * _NLANES + lax.iota(jnp.int32, _NLANES)
                rv = plsc.load_gather(r_v, [fvec, cvec])
                bits = plsc.bitcast(rv, jnp.int32)
                ks.append((bits & jnp.int32(~63)) | fvec)
            s0 = _sort_asc(ks[0])
            s1 = _sort_desc(ks[1])
            s2 = _sort_asc(ks[2])
            s3 = _sort_desc(ks[3])
            a0 = _sort_asc(jnp.minimum(s0, s1))
            a1 = _sort_asc(jnp.maximum(s0, s1))
            d0 = _sort_desc(jnp.maximum(s2, s3))
            d1 = _sort_desc(jnp.minimum(s2, s3))
            u0 = jnp.maximum(a0, d0)
            u1 = jnp.maximum(a1, d1)
            t = jnp.min(jnp.minimum(u0, u1))
            tvec = jnp.full((_NLANES,), t, jnp.int32)
            for fc in range(nfc):
                mv = jnp.where(ks[fc] >= tvec, 1.0, 0.0).astype(jnp.float32)
                fvec = fc * _NLANES + lax.iota(jnp.int32, _NLANES)
                plsc.store_scatter(m_v, [fvec, cvec], mv)
        return carry

    lax.fori_loop(0, cols // _NLANES, mask_chunk, 0)

    def start_in(i, b):
        f = i // 2
        gs = gbase + (i % 2) * gc
        pltpu.async_copy(x_hbm.at[f, pl.ds(gs, gc)], xb_v.at[b],
                         in_sem.at[b])

    def wait_in(b):
        pltpu.make_async_copy(x_hbm.at[0, pl.ds(0, gc)], xb_v.at[b],
                              in_sem.at[b]).wait()

    def start_out(i, b):
        f = i // 2
        gs = gbase + (i % 2) * gc
        pltpu.async_copy(ob_v.at[b], o_hbm.at[f, pl.ds(gs, gc)],
                         out_sem.at[b])

    def wait_out(b):
        pltpu.make_async_copy(ob_v.at[b], o_hbm.at[0, pl.ds(0, gc)],
                              out_sem.at[b]).wait()

    def compute(i, b):
        f = i // 2
        goff = (i % 2) * gc
        for gl in range(gc):
            for k in range(n2 // _NLANES):
                mk = m_v[f, pl.ds((goff + gl) * n2 + k * _NLANES, _NLANES)]
                for l in range(length):
                    xv = xb_v[b, gl, l, pl.ds(k * _NLANES, _NLANES)]
                    ob_v[b, gl, l, pl.ds(k * _NLANES, _NLANES)] = xv * mk

    for b in range(_NBUF):
        start_in(b, b)

    def slot(i, b):
        @pl.when(i >= _NBUF)
        def _():
            wait_out(b)
        wait_in(b)
        compute(i, b)
        start_out(i, b)
        @pl.when(i + _NBUF < slots)
        def _():
            start_in(i + _NBUF, b)

    def chunk(c, carry):
        i0 = c * _NBUF
        for b in range(_NBUF):
            slot(i0 + b, b)
        return carry

    lax.fori_loop(0, slots // _NBUF, chunk, 0)
    for b in range(_NBUF):
        wait_out(b)


def kernel(x):
    freq, batch, n1, n2, length = x.shape
    ncols = batch * n1 * n2
    rkey = jax.random.key(42)
    r2 = jax.random.uniform(rkey, (freq, ncols), dtype=jnp.float32)

    xt = jnp.transpose(x, (0, 1, 2, 4, 3))
    x4 = xt.reshape(freq, batch * n1, length, n2)

    mesh = plsc.VectorSubcoreMesh(core_axis_name="c", subcore_axis_name="s")
    g_per_w = (batch * n1) // 32
    sc_mul = functools.partial(
        pl.kernel,
        out_type=jax.ShapeDtypeStruct((freq, batch * n1, length, n2),
                                      jnp.float32),
        mesh=mesh,
        compiler_params=pltpu.CompilerParams(needs_layout_passes=False),
        scratch_types=[
            pltpu.VMEM((freq, g_per_w * n2), jnp.float32),
            pltpu.VMEM((freq, g_per_w * n2), jnp.float32),
            pltpu.VMEM((_NBUF, g_per_w // 2, length, n2), jnp.float32),
            pltpu.VMEM((_NBUF, g_per_w // 2, length, n2), jnp.float32),
            pltpu.SemaphoreType.DMA((_NBUF,)),
            pltpu.SemaphoreType.DMA((_NBUF,)),
        ],
    )(_sc_body)
    out = sc_mul(r2, x4)
    out5 = out.reshape(freq, batch, n1, length, n2)
    return jnp.transpose(out5, (0, 1, 2, 4, 3))


# speedup vs baseline: 1.1689x; 1.1689x over previous
import functools

import jax
import jax.numpy as jnp
from jax import lax
from jax.experimental import pallas as pl
from jax.experimental.pallas import tpu as pltpu
from jax.experimental.pallas import tpu_sc as plsc

_MASK_PERCENT = 0.5
_NBUF = 4


def _mask_body(r_ref, m_ref):
    freq, cb = r_ref.shape
    keep_thresh = float(int(_MASK_PERCENT * freq))
    bits = lax.bitcast_convert_type(r_ref[...], jnp.int32)
    fidx = lax.broadcasted_iota(jnp.int32, (freq, cb), 0)
    key = (bits & jnp.int32(~63)) | fidx
    less = key[None, :, :] < key[:, None, :]
    rank = jnp.sum(less.astype(jnp.float32), axis=1)
    m_ref[...] = (rank >= keep_thresh).astype(jnp.float32)


def _compute_mask(r2):
    freq, ncols = r2.shape
    cb = 512
    return pl.pallas_call(
        _mask_body,
        grid=(ncols // cb,),
        in_specs=[pl.BlockSpec((freq, cb), lambda g: (0, g))],
        out_specs=pl.BlockSpec((freq, cb), lambda g: (0, g)),
        out_shape=jax.ShapeDtypeStruct((freq, ncols), jnp.float32),
    )(r2)


def _sc_mul_body(m_hbm, x_hbm, o_hbm, m_v, xb_v, ob_v, in_sem, out_sem):
    freq = x_hbm.shape[0]
    length = x_hbm.shape[2]
    n2 = x_hbm.shape[3]
    nlanes = 16
    cid = lax.axis_index("c")
    sid = lax.axis_index("s")
    wid = sid * 2 + cid
    g_per_w = x_hbm.shape[1] // 32
    gbase = wid * g_per_w
    gc = g_per_w // 2
    slots = 2 * freq

    def start_in(i, b):
        f = i // 2
        gs = gbase + (i % 2) * gc
        pltpu.async_copy(x_hbm.at[f, pl.ds(gs, gc)], xb_v.at[b],
                         in_sem.at[b])

    def wait_in(b):
        pltpu.make_async_copy(x_hbm.at[0, pl.ds(0, gc)], xb_v.at[b],
                              in_sem.at[b]).wait()

    def start_out(i, b):
        f = i // 2
        gs = gbase + (i % 2) * gc
        pltpu.async_copy(ob_v.at[b], o_hbm.at[f, pl.ds(gs, gc)],
                         out_sem.at[b])

    def wait_out(b):
        pltpu.make_async_copy(ob_v.at[b], o_hbm.at[0, pl.ds(0, gc)],
                              out_sem.at[b]).wait()

    def compute(i, b):
        f = i // 2
        goff = (i % 2) * gc
        for gl in range(gc):
            for k in range(n2 // nlanes):
                mk = m_v[f, pl.ds((goff + gl) * n2 + k * nlanes, nlanes)]
                for l in range(length):
                    xv = xb_v[b, gl, l, pl.ds(k * nlanes, nlanes)]
                    ob_v[b, gl, l, pl.ds(k * nlanes, nlanes)] = xv * mk

    pltpu.sync_copy(m_hbm.at[:, pl.ds(gbase * n2, g_per_w * n2)], m_v)
    for b in range(_NBUF):
        start_in(b, b)

    def slot(i, b):
        @pl.when(i >= _NBUF)
        def _():
            wait_out(b)
        wait_in(b)
        compute(i, b)
        start_out(i, b)
        @pl.when(i + _NBUF < slots)
        def _():
            start_in(i + _NBUF, b)

    def chunk(c, carry):
        i0 = c * _NBUF
        for b in range(_NBUF):
            slot(i0 + b, b)
        return carry

    lax.fori_loop(0, slots // _NBUF, chunk, 0)
    for b in range(_NBUF):
        wait_out(b)


def kernel(x):
    freq, batch, n1, n2, length = x.shape
    ncols = batch * n1 * n2
    rkey = jax.random.key(42)
    r2 = jax.random.uniform(rkey, (freq, ncols), dtype=jnp.float32)
    m2 = _compute_mask(r2)

    xt = jnp.transpose(x, (0, 1, 2, 4, 3))
    x4 = xt.reshape(freq, batch * n1, length, n2)

    mesh = plsc.VectorSubcoreMesh(core_axis_name="c", subcore_axis_name="s")
    g_per_w = (batch * n1) // 32
    sc_mul = functools.partial(
        pl.kernel,
        out_type=jax.ShapeDtypeStruct((freq, batch * n1, length, n2),
                                      jnp.float32),
        mesh=mesh,
        scratch_types=[
            pltpu.VMEM((freq, g_per_w * n2), jnp.float32),
            pltpu.VMEM((_NBUF, g_per_w // 2, length, n2), jnp.float32),
            pltpu.VMEM((_NBUF, g_per_w // 2, length, n2), jnp.float32),
            pltpu.SemaphoreType.DMA((_NBUF,)),
            pltpu.SemaphoreType.DMA((_NBUF,)),
        ],
    )(_sc_mul_body)
    out = sc_mul(m2, x4)
    out5 = out.reshape(freq, batch, n1, length, n2)
    return jnp.transpose(out5, (0, 1, 2, 4, 3))


# speedup vs baseline: 1.1863x; 1.0149x over previous
import functools

import jax
import jax.numpy as jnp
from jax import lax
from jax.experimental import pallas as pl
from jax.experimental.pallas import tpu as pltpu
from jax.experimental.pallas import tpu_sc as plsc

_MASK_PERCENT = 0.5
_NBUF = 4


def _threefry_rounds(x0, x1, rots):
    for r in rots:
        x0 = x0 + x1
        x1 = (x1 << jnp.uint32(r)) | (x1 >> jnp.uint32(32 - r))
        x1 = x0 ^ x1
    return x0, x1


def _mask_body(m_ref, *, ncols):
    freq, cb = m_ref.shape
    keep_thresh = float(int(_MASK_PERCENT * freq))
    g = pl.program_id(0)
    fidx = lax.broadcasted_iota(jnp.int32, (freq, cb), 0)
    cidx = lax.broadcasted_iota(jnp.int32, (freq, cb), 1)
    e = (fidx * ncols + g * cb + cidx).astype(jnp.uint32)
    ks0 = jnp.uint32(0)
    ks1 = jnp.uint32(42)
    ks2 = ks0 ^ ks1 ^ jnp.uint32(0x1BD11BDA)
    x0 = jnp.full((freq, cb), ks0, jnp.uint32)
    x1 = e + ks1
    ra = (13, 15, 26, 6)
    rb = (17, 29, 16, 24)
    x0, x1 = _threefry_rounds(x0, x1, ra)
    x0 = x0 + ks1; x1 = x1 + ks2 + jnp.uint32(1)
    x0, x1 = _threefry_rounds(x0, x1, rb)
    x0 = x0 + ks2; x1 = x1 + ks0 + jnp.uint32(2)
    x0, x1 = _threefry_rounds(x0, x1, ra)
    x0 = x0 + ks0; x1 = x1 + ks1 + jnp.uint32(3)
    x0, x1 = _threefry_rounds(x0, x1, rb)
    x0 = x0 + ks1; x1 = x1 + ks2 + jnp.uint32(4)
    x0, x1 = _threefry_rounds(x0, x1, ra)
    x0 = x0 + ks2; x1 = x1 + ks0 + jnp.uint32(5)
    fb = ((x0 ^ x1) >> jnp.uint32(9)) | jnp.uint32(0x3F800000)
    r = lax.bitcast_convert_type(fb, jnp.float32) - 1.0
    bits = lax.bitcast_convert_type(r, jnp.int32)
    key = (bits & jnp.int32(~63)) | fidx
    less = key[None, :, :] < key[:, None, :]
    rank = jnp.sum(less.astype(jnp.float32), axis=1)
    m_ref[...] = (rank >= keep_thresh).astype(jnp.float32)


def _compute_mask(freq, ncols):
    cb = 512
    return pl.pallas_call(
        functools.partial(_mask_body, ncols=ncols),
        grid=(ncols // cb,),
        in_specs=[],
        out_specs=pl.BlockSpec((freq, cb), lambda g: (0, g)),
        out_shape=jax.ShapeDtypeStruct((freq, ncols), jnp.float32),
    )()


def _sc_mul_body(m_hbm, x_hbm, o_hbm, m_v, xb_v, ob_v, in_sem, out_sem):
    freq = x_hbm.shape[0]
    length = x_hbm.shape[2]
    n2 = x_hbm.shape[3]
    nlanes = 16
    cid = lax.axis_index("c")
    sid = lax.axis_index("s")
    wid = sid * 2 + cid
    g_per_w = x_hbm.shape[1] // 32
    gbase = wid * g_per_w
    gc = g_per_w // 2
    slots = 2 * freq

    def start_in(i, b):
        f = i // 2
        gs = gbase + (i % 2) * gc
        pltpu.async_copy(x_hbm.at[f, pl.ds(gs, gc)], xb_v.at[b],
                         in_sem.at[b])

    def wait_in(b):
        pltpu.make_async_copy(x_hbm.at[0, pl.ds(0, gc)], xb_v.at[b],
                              in_sem.at[b]).wait()

    def start_out(i, b):
        f = i // 2
        gs = gbase + (i % 2) * gc
        pltpu.async_copy(ob_v.at[b], o_hbm.at[f, pl.ds(gs, gc)],
                         out_sem.at[b])

    def wait_out(b):
        pltpu.make_async_copy(ob_v.at[b], o_hbm.at[0, pl.ds(0, gc)],
                              out_sem.at[b]).wait()

    def compute(i, b):
        f = i // 2
        goff = (i % 2) * gc
        for gl in range(gc):
            for k in range(n2 // nlanes):
                mk = m_v[f, pl.ds((goff + gl) * n2 + k * nlanes, nlanes)]
                for l in range(length):
                    xv = xb_v[b, gl, l, pl.ds(k * nlanes, nlanes)]
                    ob_v[b, gl, l, pl.ds(k * nlanes, nlanes)] = xv * mk

    pltpu.sync_copy(m_hbm.at[:, pl.ds(gbase * n2, g_per_w * n2)], m_v)
    for b in range(_NBUF):
        start_in(b, b)

    def slot(i, b):
        @pl.when(i >= _NBUF)
        def _():
            wait_out(b)
        wait_in(b)
        compute(i, b)
        start_out(i, b)
        @pl.when(i + _NBUF < slots)
        def _():
            start_in(i + _NBUF, b)

    def chunk(c, carry):
        i0 = c * _NBUF
        for b in range(_NBUF):
            slot(i0 + b, b)
        return carry

    lax.fori_loop(0, slots // _NBUF, chunk, 0)
    for b in range(_NBUF):
        wait_out(b)


def kernel(x):
    freq, batch, n1, n2, length = x.shape
    ncols = batch * n1 * n2
    m2 = _compute_mask(freq, ncols)

    xt = jnp.transpose(x, (0, 1, 2, 4, 3))
    x4 = xt.reshape(freq, batch * n1, length, n2)

    mesh = plsc.VectorSubcoreMesh(core_axis_name="c", subcore_axis_name="s")
    g_per_w = (batch * n1) // 32
    sc_mul = functools.partial(
        pl.kernel,
        out_type=jax.ShapeDtypeStruct((freq, batch * n1, length, n2),
                                      jnp.float32),
        mesh=mesh,
        scratch_types=[
            pltpu.VMEM((freq, g_per_w * n2), jnp.float32),
            pltpu.VMEM((_NBUF, g_per_w // 2, length, n2), jnp.float32),
            pltpu.VMEM((_NBUF, g_per_w // 2, length, n2), jnp.float32),
            pltpu.SemaphoreType.DMA((_NBUF,)),
            pltpu.SemaphoreType.DMA((_NBUF,)),
        ],
    )(_sc_mul_body)
    out = sc_mul(m2, x4)
    out5 = out.reshape(freq, batch, n1, length, n2)
    return jnp.transpose(out5, (0, 1, 2, 4, 3))
